# Initial kernel scaffold; baseline (speedup 1.0000x reference)
import jax
import jax.numpy as jnp
from jax.experimental import pallas as pl


def kernel(vid, g1, b1, g2, b2, Wqkv, bqkv, Wproj, bproj, ca0_w1, ca0_b1, ca0_w2, ca0_b2, Wr1, br1, Wr2, br2, ca1_w1, ca1_b1, ca1_w2, ca1_b2):
    raise NotImplementedError("write your pallas kernel here")


# speedup vs baseline: 7.4212x; 7.4212x over previous
import jax
import jax.numpy as jnp
from jax.experimental import pallas as pl
from jax.experimental.pallas import tpu as pltpu

T, C, H, W = 4, 96, 128, 128
WS = 5
WT = 2
K = 10
R = C // 8
HALO = WS // 2

_OFFS = [(dt, dh, dw)
         for dt in range(WT)
         for dh in range(-HALO, HALO + 1)
         for dw in range(-HALO, HALO + 1)]
NC = len(_OFFS)

HB = 16
NHT = H // HB
HWIN = HB + 2 * HALO


def _ln(x, gc, bc):
    mu = jnp.mean(x, axis=0, keepdims=True)
    var = jnp.mean((x - mu) ** 2, axis=0, keepdims=True)
    xn = (x - mu) * jax.lax.rsqrt(var + 1e-6)
    return xn * gc[:, :, None] + bc[:, :, None]


def _rollw(x, dw):
    s = dw % W
    if s == 0:
        return x
    return jnp.roll(x, s, axis=-1)


def _p1_body(vid_ref, g1_ref, b1_ref, wqkv_ref, bqkv_ref, q_ref, k_ref, v_ref):
    x = vid_ref[0]
    xn = _ln(x, g1_ref[...], b1_ref[...])
    xm = xn.reshape(C, HB * W)
    qkv = jnp.dot(wqkv_ref[...], xm, preferred_element_type=jnp.float32)
    qkv = qkv + bqkv_ref[...]
    q_ref[0] = qkv[0:C].reshape(C, HB, W)
    k_ref[0] = qkv[C:2 * C].reshape(C, HB, W)
    v_ref[0] = qkv[2 * C:3 * C].reshape(C, HB, W)


def _p2_body(q_ref, k0_ref, k1_ref, v0_ref, v1_ref, wp_ref, bp_ref,
             out_ref, pool_ref):
    q = q_ref[0]
    scale = 1.0 / float(C) ** 0.5
    dists = []
    for (dt, dh, dw) in _OFFS:
        kb = (k0_ref if dt == 0 else k1_ref)[0, 0]
        ks = _rollw(kb[:, HALO - dh:HALO - dh + HB, :], dw)
        dists.append(jnp.sum(q * ks, axis=0) * scale)
    d = jnp.stack(dists, axis=0)

    run = d
    kept = jnp.zeros(d.shape, jnp.bool_)
    ci_arr = jax.lax.broadcasted_iota(jnp.int32, d.shape, 0)
    mx = None
    for i in range(K):
        m = jnp.max(run, axis=0, keepdims=True)
        if i == 0:
            mx = m
        eq = run == m
        sel = jnp.where(eq, ci_arr, NC)
        mi = jnp.min(sel, axis=0, keepdims=True)
        first = jnp.logical_and(eq, ci_arr == mi)
        kept = jnp.logical_or(kept, first)
        run = jnp.where(first, -1e30, run)
    w = jnp.where(kept, jnp.exp(d - mx), 0.0)
    w = w / jnp.sum(w, axis=0, keepdims=True)

    acc = None
    for ci, (dt, dh, dw) in enumerate(_OFFS):
        vb = (v0_ref if dt == 0 else v1_ref)[0, 0]
        vs = _rollw(vb[:, HALO - dh:HALO - dh + HB, :], dw)
        term = w[ci][None, :, :] * vs
        acc = term if acc is None else acc + term
    o = jnp.dot(wp_ref[...], acc.reshape(C, HB * W),
                preferred_element_type=jnp.float32) + bp_ref[...]
    out_ref[0] = o.reshape(C, HB, W)
    pool_ref[0, 0] = jnp.sum(o, axis=1, keepdims=True)


def _gate(pool, w1_ref, b1_ref, w2_ref, b2_ref):
    p = jnp.sum(pool, axis=0) * (1.0 / (H * W))
    h = jnp.maximum(jnp.dot(w1_ref[...], p,
                            preferred_element_type=jnp.float32)
                    + b1_ref[...], 0.0)
    return jax.nn.sigmoid(jnp.dot(w2_ref[...], h,
                                  preferred_element_type=jnp.float32)
                          + b2_ref[...])


def _p3_body(vid_ref, out_ref, pool0_ref, w1_ref, b1_ref, w2_ref, b2_ref,
             g2_ref, b2c_ref, wr1_ref, br1_ref, wr2_ref, br2_ref,
             x2_ref, pool1_ref):
    s0 = _gate(pool0_ref[0], w1_ref, b1_ref, w2_ref, b2_ref)
    x = vid_ref[0] + out_ref[0] * s0[:, :, None]
    xn = _ln(x, g2_ref[...], b2c_ref[...])
    xm = xn.reshape(C, HB * W)
    r = jnp.maximum(jnp.dot(wr1_ref[...], xm,
                            preferred_element_type=jnp.float32)
                    + br1_ref[...], 0.0)
    r = jnp.dot(wr2_ref[...], r,
                preferred_element_type=jnp.float32) + br2_ref[...]
    x2 = xm + r
    x2_ref[0] = x2.reshape(C, HB, W)
    pool1_ref[0, 0] = jnp.sum(x2, axis=1, keepdims=True)


def _p4_body(x2_ref, pool1_ref, w1_ref, b1_ref, w2_ref, b2_ref, y_ref):
    s1 = _gate(pool1_ref[0], w1_ref, b1_ref, w2_ref, b2_ref)
    y_ref[0] = x2_ref[0] * s1[:, :, None]


def _tile_spec():
    return pl.BlockSpec((1, C, HB, W), lambda t, h: (t, 0, h, 0))


def _full(shape):
    return pl.BlockSpec(shape, lambda t, h: tuple(0 for _ in shape))


def kernel(vid, g1, b1, g2, b2, Wqkv, bqkv, Wproj, bproj,
           ca0_w1, ca0_b1, ca0_w2, ca0_b2, Wr1, br1, Wr2, br2,
           ca1_w1, ca1_b1, ca1_w2, ca1_b2):
    f32 = jnp.float32
    x = vid[0]
    col = lambda a: a.reshape(-1, 1).astype(f32)
    g1c, b1c, g2c, b2c = col(g1), col(b1), col(g2), col(b2)
    bqkvc, bprojc, br1c, br2c = col(bqkv), col(bproj), col(br1), col(br2)
    cb0_1, cb0_2, cb1_1, cb1_2 = col(ca0_b1), col(ca0_b2), col(ca1_b1), col(ca1_b2)

    grid = (T, NHT)
    params = pltpu.CompilerParams(
        dimension_semantics=("parallel", "parallel"))

    qkv_shape = jax.ShapeDtypeStruct((T, C, H, W), f32)
    q, k, v = pl.pallas_call(
        _p1_body,
        grid=grid,
        in_specs=[_tile_spec(), _full((C, 1)), _full((C, 1)),
                  _full((3 * C, C)), _full((3 * C, 1))],
        out_specs=[_tile_spec(), _tile_spec(), _tile_spec()],
        out_shape=[qkv_shape, qkv_shape, qkv_shape],
        compiler_params=params,
    )(x, g1c, b1c, Wqkv.astype(f32), bqkvc)

    def windows(a):
        ap = jnp.concatenate([a[:, :, H - HALO:, :], a, a[:, :, :HALO, :]],
                             axis=2)
        return jnp.stack([ap[:, :, i * HB:i * HB + HWIN, :]
                          for i in range(NHT)], axis=1)

    kw = windows(k)
    vw = windows(v)

    win_spec0 = pl.BlockSpec((1, 1, C, HWIN, W), lambda t, h: (t, h, 0, 0, 0))
    win_spec1 = pl.BlockSpec((1, 1, C, HWIN, W),
                             lambda t, h: ((t + T - 1) % T, h, 0, 0, 0))
    pool_spec = pl.BlockSpec((1, 1, C, 1), lambda t, h: (t, h, 0, 0))

    out, pool0 = pl.pallas_call(
        _p2_body,
        grid=grid,
        in_specs=[_tile_spec(), win_spec0, win_spec1, win_spec0, win_spec1,
                  _full((C, C)), _full((C, 1))],
        out_specs=[_tile_spec(), pool_spec],
        out_shape=[qkv_shape, jax.ShapeDtypeStruct((T, NHT, C, 1), f32)],
        compiler_params=params,
    )(q, kw, kw, vw, vw, Wproj.astype(f32), bprojc)

    pool_in_spec = pl.BlockSpec((1, NHT, C, 1), lambda t, h: (t, 0, 0, 0))

    x2, pool1 = pl.pallas_call(
        _p3_body,
        grid=grid,
        in_specs=[_tile_spec(), _tile_spec(), pool_in_spec,
                  _full((R, C)), _full((R, 1)), _full((C, R)), _full((C, 1)),
                  _full((C, 1)), _full((C, 1)),
                  _full((C, C)), _full((C, 1)), _full((C, C)), _full((C, 1))],
        out_specs=[_tile_spec(), pool_spec],
        out_shape=[qkv_shape, jax.ShapeDtypeStruct((T, NHT, C, 1), f32)],
        compiler_params=params,
    )(x, out, pool0, ca0_w1.astype(f32), cb0_1, ca0_w2.astype(f32), cb0_2,
      g2c, b2c, Wr1.astype(f32), br1c, Wr2.astype(f32), br2c)

    y = pl.pallas_call(
        _p4_body,
        grid=grid,
        in_specs=[_tile_spec(), pool_in_spec,
                  _full((R, C)), _full((R, 1)), _full((C, R)), _full((C, 1))],
        out_specs=_tile_spec(),
        out_shape=qkv_shape,
        compiler_params=params,
    )(x2, pool1, ca1_w1.astype(f32), cb1_1, ca1_w2.astype(f32), cb1_2)

    return y[None]


# speedup vs baseline: 9.1245x; 1.2295x over previous
import jax
import jax.numpy as jnp
from jax.experimental import pallas as pl
from jax.experimental.pallas import tpu as pltpu

T, C, H, W = 4, 96, 128, 128
WS = 5
WT = 2
K = 10
R = C // 8
HALO = WS // 2

_OFFS = [(dt, dh, dw)
         for dt in range(WT)
         for dh in range(-HALO, HALO + 1)
         for dw in range(-HALO, HALO + 1)]
NC = len(_OFFS)

HB = 16
NHT = H // HB
HWIN = HB + 2 * HALO
BF = jnp.bfloat16


def _ln(x, gc, bc):
    mu = jnp.mean(x, axis=0, keepdims=True)
    var = jnp.mean((x - mu) ** 2, axis=0, keepdims=True)
    xn = (x - mu) * jax.lax.rsqrt(var + 1e-6)
    return xn * gc[:, :, None] + bc[:, :, None]


def _rollw(x, dw):
    s = dw % W
    if s == 0:
        return x
    return pltpu.roll(x, s, axis=x.ndim - 1)


def _p1_body(vid_ref, g1_ref, b1_ref, wqkv_ref, bqkv_ref,
             q_ref, k_ref, v_ref, kl_ref, kf_ref, vl_ref, vf_ref):
    x = vid_ref[0]
    xn = _ln(x, g1_ref[...], b1_ref[...]).astype(BF)
    qkv = jnp.dot(wqkv_ref[...], xn.reshape(C, HB * W),
                  preferred_element_type=jnp.float32) + bqkv_ref[...]
    qkv = qkv.reshape(3 * C, HB, W)
    q_ref[0] = qkv[0:C].astype(BF)
    k = qkv[C:2 * C].astype(BF)
    v = qkv[2 * C:3 * C].astype(BF)
    k_ref[0] = k
    v_ref[0] = v
    kl_ref[0, 0] = k[:, HB - HALO:, :]
    kf_ref[0, 0] = k[:, :HALO, :]
    vl_ref[0, 0] = v[:, HB - HALO:, :]
    vf_ref[0, 0] = v[:, :HALO, :]


def _p2_body(q_ref, kc0_ref, kl0_ref, kf0_ref, kc1_ref, kl1_ref, kf1_ref,
             vc0_ref, vl0_ref, vf0_ref, vc1_ref, vl1_ref, vf1_ref,
             wp_ref, bp_ref, out_ref, pool_ref):
    scale = 1.0 / float(C) ** 0.5
    q = q_ref[0].astype(jnp.float32) * scale

    def window(c_ref, l_ref, f_ref):
        return jnp.concatenate(
            [l_ref[0, 0], c_ref[0], f_ref[0, 0]], axis=1).astype(jnp.float32)

    k0 = window(kc0_ref, kl0_ref, kf0_ref)
    k1 = window(kc1_ref, kl1_ref, kf1_ref)
    v0 = window(vc0_ref, vl0_ref, vf0_ref)
    v1 = window(vc1_ref, vl1_ref, vf1_ref)

    kslc = {(dt, dh): (k0 if dt == 0 else k1)[:, HALO - dh:HALO - dh + HB, :]
            for dt in range(WT) for dh in range(-HALO, HALO + 1)}
    vslc = {(dt, dh): (v0 if dt == 0 else v1)[:, HALO - dh:HALO - dh + HB, :]
            for dt in range(WT) for dh in range(-HALO, HALO + 1)}

    dists = []
    for (dt, dh, dw) in _OFFS:
        ks = _rollw(kslc[(dt, dh)], dw)
        dists.append(jnp.sum(q * ks, axis=0))
    d = jnp.stack(dists, axis=0)

    run = d
    ci_arr = jax.lax.broadcasted_iota(jnp.int32, d.shape, 0)
    mx = None
    for i in range(K):
        m = jnp.max(run, axis=0, keepdims=True)
        if i == 0:
            mx = m
        sel = jnp.where(run == m, ci_arr, NC)
        mi = jnp.min(sel, axis=0, keepdims=True)
        run = jnp.where(ci_arr == mi, -1e30, run)
    kept = run <= -1e29
    w = jnp.where(kept, jnp.exp(d - mx), 0.0)
    w = w / jnp.sum(w, axis=0, keepdims=True)

    acc = None
    for ci, (dt, dh, dw) in enumerate(_OFFS):
        vs = _rollw(vslc[(dt, dh)], dw)
        term = w[ci][None, :, :] * vs
        acc = term if acc is None else acc + term
    o = jnp.dot(wp_ref[...], acc.astype(BF).reshape(C, HB * W),
                preferred_element_type=jnp.float32) + bp_ref[...]
    out_ref[0] = o.reshape(C, HB, W)
    pool_ref[0, 0] = jnp.sum(o, axis=1, keepdims=True)


def _gate(pool, w1_ref, b1_ref, w2_ref, b2_ref):
    p = jnp.sum(pool, axis=0) * (1.0 / (H * W))
    h = jnp.maximum(jnp.dot(w1_ref[...], p,
                            preferred_element_type=jnp.float32)
                    + b1_ref[...], 0.0)
    return jax.nn.sigmoid(jnp.dot(w2_ref[...], h,
                                  preferred_element_type=jnp.float32)
                          + b2_ref[...])


def _p3_body(vid_ref, out_ref, pool0_ref, w1_ref, b1_ref, w2_ref, b2_ref,
             g2_ref, b2c_ref, wr1_ref, br1_ref, wr2_ref, br2_ref,
             x2_ref, pool1_ref):
    s0 = _gate(pool0_ref[0], w1_ref, b1_ref, w2_ref, b2_ref)
    x = vid_ref[0] + out_ref[0] * s0[:, :, None]
    xn = _ln(x, g2_ref[...], b2c_ref[...])
    xm = xn.reshape(C, HB * W)
    r = jnp.maximum(jnp.dot(wr1_ref[...], xm.astype(BF),
                            preferred_element_type=jnp.float32)
                    + br1_ref[...], 0.0)
    r = jnp.dot(wr2_ref[...], r.astype(BF),
                preferred_element_type=jnp.float32) + br2_ref[...]
    x2 = xm + r
    x2_ref[0] = x2.reshape(C, HB, W)
    pool1_ref[0, 0] = jnp.sum(x2, axis=1, keepdims=True)


def _p4_body(x2_ref, pool1_ref, w1_ref, b1_ref, w2_ref, b2_ref, y_ref):
    s1 = _gate(pool1_ref[0], w1_ref, b1_ref, w2_ref, b2_ref)
    y_ref[0] = x2_ref[0] * s1[:, :, None]


def _tile_spec():
    return pl.BlockSpec((1, C, HB, W), lambda t, h: (t, 0, h, 0))


def _full(shape):
    return pl.BlockSpec(shape, lambda t, h: tuple(0 for _ in shape))


def kernel(vid, g1, b1, g2, b2, Wqkv, bqkv, Wproj, bproj,
           ca0_w1, ca0_b1, ca0_w2, ca0_b2, Wr1, br1, Wr2, br2,
           ca1_w1, ca1_b1, ca1_w2, ca1_b2):
    f32 = jnp.float32
    x = vid[0]
    col = lambda a: a.reshape(-1, 1).astype(f32)
    g1c, b1c, g2c, b2c = col(g1), col(b1), col(g2), col(b2)
    bqkvc, bprojc, br1c, br2c = col(bqkv), col(bproj), col(br1), col(br2)
    cb0_1, cb0_2, cb1_1, cb1_2 = col(ca0_b1), col(ca0_b2), col(ca1_b1), col(ca1_b2)

    grid = (T, NHT)
    params = pltpu.CompilerParams(
        dimension_semantics=("parallel", "parallel"))

    arr_shape = jax.ShapeDtypeStruct((T, C, H, W), f32)
    bf_shape = jax.ShapeDtypeStruct((T, C, H, W), BF)
    edge_shape = jax.ShapeDtypeStruct((T, NHT, C, HALO, W), BF)
    edge_out = pl.BlockSpec((1, 1, C, HALO, W), lambda t, h: (t, h, 0, 0, 0))

    q, k, v, kl, kf, vl, vf = pl.pallas_call(
        _p1_body,
        grid=grid,
        in_specs=[_tile_spec(), _full((C, 1)), _full((C, 1)),
                  _full((3 * C, C)), _full((3 * C, 1))],
        out_specs=[_tile_spec(), _tile_spec(), _tile_spec(),
                   edge_out, edge_out, edge_out, edge_out],
        out_shape=[bf_shape, bf_shape, bf_shape,
                   edge_shape, edge_shape, edge_shape, edge_shape],
        compiler_params=params,
    )(x, g1c, b1c, Wqkv.astype(BF), bqkvc)

    halo = (1, 1, C, HALO, W)
    tb = (1, C, HB, W)
    c0 = pl.BlockSpec(tb, lambda t, h: (t, 0, h, 0))
    l0 = pl.BlockSpec(halo, lambda t, h: (t, (h + NHT - 1) % NHT, 0, 0, 0))
    f0 = pl.BlockSpec(halo, lambda t, h: (t, (h + 1) % NHT, 0, 0, 0))
    c1 = pl.BlockSpec(tb, lambda t, h: ((t + T - 1) % T, 0, h, 0))
    l1 = pl.BlockSpec(halo, lambda t, h: ((t + T - 1) % T,
                                          (h + NHT - 1) % NHT, 0, 0, 0))
    f1 = pl.BlockSpec(halo, lambda t, h: ((t + T - 1) % T,
                                          (h + 1) % NHT, 0, 0, 0))
    pool_spec = pl.BlockSpec((1, 1, C, 1), lambda t, h: (t, h, 0, 0))
    pool_shape = jax.ShapeDtypeStruct((T, NHT, C, 1), f32)

    out, pool0 = pl.pallas_call(
        _p2_body,
        grid=grid,
        in_specs=[c0, c0, l0, f0, c1, l1, f1,
                  c0, l0, f0, c1, l1, f1,
                  _full((C, C)), _full((C, 1))],
        out_specs=[_tile_spec(), pool_spec],
        out_shape=[arr_shape, pool_shape],
        compiler_params=params,
    )(q, k, kl, kf, k, kl, kf, v, vl, vf, v, vl, vf,
      Wproj.astype(BF), bprojc)

    pool_in_spec = pl.BlockSpec((1, NHT, C, 1), lambda t, h: (t, 0, 0, 0))

    x2, pool1 = pl.pallas_call(
        _p3_body,
        grid=grid,
        in_specs=[_tile_spec(), _tile_spec(), pool_in_spec,
                  _full((R, C)), _full((R, 1)), _full((C, R)), _full((C, 1)),
                  _full((C, 1)), _full((C, 1)),
                  _full((C, C)), _full((C, 1)), _full((C, C)), _full((C, 1))],
        out_specs=[_tile_spec(), pool_spec],
        out_shape=[arr_shape, pool_shape],
        compiler_params=params,
    )(x, out, pool0, ca0_w1.astype(f32), cb0_1, ca0_w2.astype(f32), cb0_2,
      g2c, b2c, Wr1.astype(BF), br1c, Wr2.astype(BF), br2c)

    y = pl.pallas_call(
        _p4_body,
        grid=grid,
        in_specs=[_tile_spec(), pool_in_spec,
                  _full((R, C)), _full((R, 1)), _full((C, R)), _full((C, 1))],
        out_specs=_tile_spec(),
        out_shape=arr_shape,
        compiler_params=params,
    )(x2, pool1, ca1_w1.astype(f32), cb1_1, ca1_w2.astype(f32), cb1_2)

    return y[None]
